# baseline (speedup 1.0000x reference)
import functools

import jax
import jax.numpy as jnp
from jax import lax
from jax.experimental import pallas as pl
from jax.experimental.pallas import tpu as pltpu
from jax.experimental.pallas import tpu_sc as plsc

_B = 16384
_E = 16


@jax.jit
def _sc_gather3(cat_idx, store_idx, pa_idx, cat_table, store_table, pa_table):
    info = plsc.get_sparse_core_info()
    nc, ns = info.num_cores, info.num_subcores
    nw = nc * ns
    bpw = _B // nw

    mesh = plsc.VectorSubcoreMesh(core_axis_name="c", subcore_axis_name="s")

    @functools.partial(
        pl.kernel,
        mesh=mesh,
        out_type=[jax.ShapeDtypeStruct((_B, _E), jnp.float32)] * 3,
        scratch_types=[
            pltpu.VMEM((bpw,), jnp.int32),
            pltpu.VMEM((bpw,), jnp.int32),
            pltpu.VMEM((bpw,), jnp.int32),
            pltpu.VMEM((bpw, _E), jnp.float32),
            pltpu.VMEM((bpw, _E), jnp.float32),
            pltpu.VMEM((bpw, _E), jnp.float32),
            pltpu.SemaphoreType.DMA,
            pltpu.SemaphoreType.DMA,
            pltpu.SemaphoreType.DMA,
        ],
        compiler_params=pltpu.CompilerParams(use_tc_tiling_on_sc=False),
    )
    def gather_kernel(cat_i, store_i, pa_i, cat_t, store_t, pa_t,
                      cat_o, store_o, pa_o,
                      idx0, idx1, idx2, rows0, rows1, rows2, s0, s1, s2):
        wid = lax.axis_index("s") * nc + lax.axis_index("c")
        base = wid * bpw
        pltpu.sync_copy(cat_i.at[pl.ds(base, bpw)], idx0)
        pltpu.sync_copy(store_i.at[pl.ds(base, bpw)], idx1)
        pltpu.sync_copy(pa_i.at[pl.ds(base, bpw)], idx2)
        c0 = pltpu.async_copy(cat_t.at[idx0], rows0, s0)
        c1 = pltpu.async_copy(store_t.at[idx1], rows1, s1)
        c2 = pltpu.async_copy(pa_t.at[idx2], rows2, s2)
        c0.wait()
        c1.wait()
        c2.wait()
        pltpu.sync_copy(rows0, cat_o.at[pl.ds(base, bpw)])
        pltpu.sync_copy(rows1, store_o.at[pl.ds(base, bpw)])
        pltpu.sync_copy(rows2, pa_o.at[pl.ds(base, bpw)])

    return gather_kernel(cat_idx, store_idx, pa_idx,
                         cat_table, store_table, pa_table)


def _dense_body(cat_ref, store_ref, pa_ref, num_ref, title_ref,
                wn_ref, bn_ref, wt_ref, bt_ref, wo_ref, bo_ref, out_ref):
    wo = wo_ref[...]
    acc = jnp.dot(cat_ref[...], wo[0:16, :], preferred_element_type=jnp.float32)
    acc += jnp.dot(store_ref[...], wo[16:32, :], preferred_element_type=jnp.float32)
    acc += jnp.dot(pa_ref[...], wo[32:48, :], preferred_element_type=jnp.float32)
    nf = jnp.dot(num_ref[...], wn_ref[...], preferred_element_type=jnp.float32)
    nf += bn_ref[...]
    acc += jnp.dot(nf, wo[48:64, :], preferred_element_type=jnp.float32)
    te = jnp.dot(title_ref[...], wt_ref[...], preferred_element_type=jnp.float32)
    te += bt_ref[...]
    acc += jnp.dot(te, wo[64:128, :], preferred_element_type=jnp.float32)
    out_ref[...] = acc + bo_ref[...]


@jax.jit
def _tc_dense(cat_e, store_e, pa_e, num_pad, title, WnT, bn2, WtT, bt2, WoT, bo2):
    R = 2048
    grid = (_B // R,)
    row_blk = lambda i: (i, 0)
    full = lambda i: (0, 0)
    return pl.pallas_call(
        _dense_body,
        grid=grid,
        in_specs=[
            pl.BlockSpec((R, 16), row_blk),
            pl.BlockSpec((R, 16), row_blk),
            pl.BlockSpec((R, 16), row_blk),
            pl.BlockSpec((R, 8), row_blk),
            pl.BlockSpec((R, 384), row_blk),
            pl.BlockSpec((8, 16), full),
            pl.BlockSpec((1, 16), full),
            pl.BlockSpec((384, 64), full),
            pl.BlockSpec((1, 64), full),
            pl.BlockSpec((128, 128), full),
            pl.BlockSpec((1, 128), full),
        ],
        out_specs=pl.BlockSpec((R, 128), row_blk),
        out_shape=jax.ShapeDtypeStruct((_B, 128), jnp.float32),
        compiler_params=pltpu.CompilerParams(
            dimension_semantics=("arbitrary",),
        ),
    )(cat_e, store_e, pa_e, num_pad, title, WnT, bn2, WtT, bt2, WoT, bo2)


def kernel(category, store, parent_asin, numeric_features, title_embedding,
           cat_table, store_table, pa_table, Wn, bn, Wt, bt, Wo, bo):
    cat_e, store_e, pa_e = _sc_gather3(
        category.astype(jnp.int32), store.astype(jnp.int32),
        parent_asin.astype(jnp.int32), cat_table, store_table, pa_table)
    num_pad = jnp.pad(numeric_features, ((0, 0), (0, 5)))
    WnT = jnp.pad(Wn.T, ((0, 5), (0, 0)))
    return _tc_dense(
        cat_e, store_e, pa_e, num_pad, title_embedding,
        WnT, bn.reshape(1, 16), Wt.T, bt.reshape(1, 64),
        Wo.T, bo.reshape(1, 128))
